# baseline (speedup 1.0000x reference)
import functools

import jax
import jax.numpy as jnp
from jax import lax
from jax.experimental import pallas as pl
from jax.experimental.pallas import tpu as pltpu
from jax.experimental.pallas import tpu_sc as plsc

_EPS = 0.1
_SCALE = _EPS / (1.0 - _EPS * _EPS)
_BND = 1.0 - 2.0 * _EPS
_HALF_PI = 1.5707963267948966

_NC, _NS, _L = 2, 16, 16
_NW = _NC * _NS
_B = 16384
_D = 64
_PPW = _B // _NW
_RPW = 2 * _PPW
_CHUNK = 128
_NCH = _RPW // _CHUNK
_G = _PPW // _L


def _rsqrt(x):
  i = plsc.bitcast(x, jnp.int32)
  y = plsc.bitcast(jnp.int32(0x5F3759DF) - (i >> 1), jnp.float32)
  for _ in range(3):
    y = y * (1.5 - 0.5 * x * y * y)
  return y


def _sqrt(x):
  return x * _rsqrt(jnp.maximum(x, 1e-30))


def _asin(x):
  a = jnp.abs(x)
  big = a > 0.5
  zb = 0.5 * (1.0 - a)
  z = jnp.where(big, zb, a * a)
  w = jnp.where(big, _sqrt(zb), a)
  p = ((((4.2163199048e-2 * z + 2.4181311049e-2) * z + 4.5470025998e-2) * z
        + 7.4953002686e-2) * z + 1.6666752422e-1)
  p = w + w * z * p
  r = jnp.where(big, _HALF_PI - 2.0 * p, p)
  return jnp.where(x < 0.0, -r, r)


def _sigmoid(x):
  return 1.0 / (1.0 + jnp.exp(-x))


def _body(idx_hbm, ang_hbm, rad_hbm, out_hbm, idx_v, rows_v, rr_v, out_v, sem):
  w = lax.axis_index("s") * _NC + lax.axis_index("c")
  row_base = w * _RPW

  pltpu.sync_copy(idx_hbm.at[pl.ds(row_base, _RPW)], idx_v)

  copies = []
  for c in range(_NCH):
    sl = pl.ds(c * _CHUNK, _CHUNK)
    copies.append(pltpu.async_copy(ang_hbm.at[idx_v.at[sl]], rows_v.at[sl], sem))
    copies.append(pltpu.async_copy(rad_hbm.at[idx_v.at[sl]], rr_v.at[sl], sem))
  for cp in copies:
    cp.wait()

  lanes = lax.iota(jnp.int32, _L)

  def group_body(g, carry):
    u_rows = g * (2 * _L) + 2 * lanes
    v_rows = u_rows + 1

    def d_body(dd, acc):
      su, sv, duv = acc
      dcol = jnp.full((_L,), dd, dtype=jnp.int32)
      uu = plsc.load_gather(rows_v, [u_rows, dcol])
      vv = plsc.load_gather(rows_v, [v_rows, dcol])
      return su + uu * uu, sv + vv * vv, duv + uu * vv

    zero = jnp.zeros((_L,), jnp.float32)
    su, sv, duv = lax.fori_loop(0, _D, d_body, (zero, zero, zero))

    r0 = _EPS + _BND * _sigmoid(plsc.load_gather(rr_v, [u_rows]))
    r1 = _EPS + _BND * _sigmoid(plsc.load_gather(rr_v, [v_rows]))
    r20 = r0 * r0
    r21 = r1 * r1

    inv_nu = _rsqrt(jnp.maximum(su, 1e-24))
    inv_nv = _rsqrt(jnp.maximum(sv, 1e-24))
    dot = r0 * r1 * (duv * inv_nu * inv_nv)
    hu = su * inv_nu * inv_nu
    hv = sv * inv_nv * inv_nv
    dist = _sqrt(jnp.maximum(r20 * hu + r21 * hv - 2.0 * dot, 0.0))

    ps = _SCALE * (1.0 - r20) / r0
    ps = jnp.minimum(jnp.maximum(ps, -1.0), 1.0)
    pa = _asin(ps)

    cn = dot * (1.0 + r20) - r20 * (1.0 + r21)
    cd = r0 * dist * _sqrt(jnp.maximum(1.0 + r20 * r21 - 2.0 * dot, 0.0)) + 1e-22
    cosv = jnp.minimum(jnp.maximum(cn / cd, -1.0), 1.0)

    res = jnp.minimum(pa + _asin(cosv) - _HALF_PI, 0.0)
    plsc.store_scatter(out_v, [g * _L + lanes], res)
    return carry

  lax.fori_loop(0, _G, group_body, 0)
  pltpu.sync_copy(out_v, out_hbm.at[pl.ds(w * _PPW, _PPW)])


_sc_call = functools.partial(
    pl.kernel,
    mesh=plsc.VectorSubcoreMesh(core_axis_name="c", subcore_axis_name="s"),
    out_type=jax.ShapeDtypeStruct((_B,), jnp.float32),
    scratch_types=[
        pltpu.VMEM((_RPW,), jnp.int32),
        pltpu.VMEM((_RPW, _D), jnp.float32),
        pltpu.VMEM((_RPW,), jnp.float32),
        pltpu.VMEM((_PPW,), jnp.float32),
        pltpu.SemaphoreType.DMA,
    ],
    compiler_params=pltpu.CompilerParams(
        use_tc_tiling_on_sc=False, needs_layout_passes=False),
)(_body)


def kernel(idxs, angles_w, radii_raw):
  idx_flat = idxs.reshape(-1).astype(jnp.int32)
  return _sc_call(idx_flat, angles_w, radii_raw)
